# Initial kernel scaffold; baseline (speedup 1.0000x reference)
import jax
import jax.numpy as jnp
from jax.experimental import pallas as pl


def kernel(x, edge_index, W1, b1, W2, b2, W3, b3, g1, be1, g2, be2, g3, be3, Wlin, blin):
    raise NotImplementedError("write your pallas kernel here")


# speedup vs baseline: 9.0071x; 9.0071x over previous
import functools

import jax
import jax.numpy as jnp
from jax import lax
from jax.experimental import pallas as pl
from jax.experimental.pallas import tpu as pltpu
from jax.experimental.pallas import tpu_sc as plsc

_N = 10000
_E = 320000
_D = 128
_H = 128
_C = 10
_EPS = 1e-5

_NC = 2
_NS = 16
_NW = _NC * _NS

_CH = 128
_CHUNKS = 79
_EW = _CH * _CHUNKS
_EPAD = _NW * _EW
_NPAD = 10240
_RPT = _NPAD // _NS


def _mesh():
    return plsc.VectorSubcoreMesh(core_axis_name="c", subcore_axis_name="s")


@functools.partial(
    pl.kernel,
    mesh=_mesh(),
    out_type=jax.ShapeDtypeStruct((_NC, _NPAD, 16), jnp.float32),
    scratch_types=[
        pltpu.VMEM((_CH, 16), jnp.float32),
        pltpu.VMEM((1, _CH), jnp.int32),
        pltpu.VMEM_SHARED((_NPAD, 16), jnp.float32),
    ],
)
def _deg_kernel(dst_hbm, out_hbm, onesb, dstb, acc):
    c = lax.axis_index("c")
    s = lax.axis_index("s")
    wid = s * _NC + c

    def _fill(val):
        def b(i, _):
            onesb[i] = jnp.full((16,), val, jnp.float32)
            return 0
        lax.fori_loop(0, _CH, b, 0)

    _fill(0.0)
    r0 = s * _RPT

    def zb(k, _):
        pltpu.sync_copy(onesb, acc.at[pl.ds(r0 + k * _CH, _CH)])
        return 0

    lax.fori_loop(0, _RPT // _CH, zb, 0)
    _fill(1.0)
    plsc.subcore_barrier()

    ebase = wid * _EW

    def body(j, _):
        pltpu.sync_copy(dst_hbm.at[pl.ds(ebase + j * _CH, _CH)], dstb.at[0])
        pltpu.sync_copy(onesb, acc.at[dstb.at[0]], add=True)
        return 0

    lax.fori_loop(0, _CHUNKS, body, 0)
    plsc.subcore_barrier()
    pltpu.sync_copy(acc.at[pl.ds(r0, _RPT)], out_hbm.at[c, pl.ds(r0, _RPT)])


@functools.partial(
    pl.kernel,
    mesh=_mesh(),
    out_type=jax.ShapeDtypeStruct((_NC, _NPAD, _D), jnp.float32),
    scratch_types=[
        pltpu.VMEM((_CH, _D), jnp.float32),
        pltpu.VMEM((1, _CH), jnp.int32),
        pltpu.VMEM((1, _CH), jnp.int32),
        pltpu.VMEM_SHARED((_NPAD, _D), jnp.float32),
        pltpu.SemaphoreType.DMA,
    ],
)
def _scatter_kernel(y_hbm, src_hbm, dst_hbm, out_hbm, rows, srcb, dstb, acc, sem):
    c = lax.axis_index("c")
    s = lax.axis_index("s")
    wid = s * _NC + c

    def zr(i, _):
        r = i // (_D // 16)
        k = i % (_D // 16)
        rows[r, pl.ds(k * 16, 16)] = jnp.zeros((16,), jnp.float32)
        return 0

    lax.fori_loop(0, _CH * (_D // 16), zr, 0)
    r0 = s * _RPT

    def zb(k, _):
        pltpu.sync_copy(rows, acc.at[pl.ds(r0 + k * _CH, _CH)])
        return 0

    lax.fori_loop(0, _RPT // _CH, zb, 0)
    plsc.subcore_barrier()

    ebase = wid * _EW

    def body(j, _):
        off = ebase + j * _CH
        pltpu.sync_copy(src_hbm.at[pl.ds(off, _CH)], srcb.at[0])
        pltpu.sync_copy(dst_hbm.at[pl.ds(off, _CH)], dstb.at[0])
        pltpu.async_copy(y_hbm.at[srcb.at[0]], rows, sem).wait()
        pltpu.sync_copy(rows, acc.at[dstb.at[0]], add=True)
        return 0

    lax.fori_loop(0, _CHUNKS, body, 0)
    plsc.subcore_barrier()
    pltpu.sync_copy(acc.at[pl.ds(r0, _RPT)], out_hbm.at[c, pl.ds(r0, _RPT)])


def _dinv_from(dp):
    deg = dp[0, 0:_N, 0:1] + dp[1, 0:_N, 0:1] + 1.0
    return lax.rsqrt(deg)


def _tc_first(degp, x, w):
    def body(degp_ref, x_ref, w_ref, y_ref):
        dinv = _dinv_from(degp_ref[...])
        xw = jnp.dot(x_ref[...], w_ref[...], preferred_element_type=jnp.float32)
        y_ref[...] = xw * dinv

    return pl.pallas_call(
        body, out_shape=jax.ShapeDtypeStruct((_N, _D), jnp.float32)
    )(degp, x, w)


def _tc_mid(degp, p, y, b, g, be, w):
    def body(degp_ref, p_ref, y_ref, b_ref, g_ref, be_ref, w_ref, out_ref):
        dinv = _dinv_from(degp_ref[...])
        pv = p_ref[...]
        t = (pv[0, 0:_N, :] + pv[1, 0:_N, :] + y_ref[...]) * dinv + b_ref[...]
        t = jnp.maximum(t, 0.0)
        m = jnp.mean(t, axis=0, keepdims=True)
        v = jnp.mean((t - m) * (t - m), axis=0, keepdims=True)
        h = g_ref[...] * (t - m) * lax.rsqrt(v + _EPS) + be_ref[...]
        out_ref[...] = (
            jnp.dot(h, w_ref[...], preferred_element_type=jnp.float32) * dinv
        )

    return pl.pallas_call(
        body, out_shape=jax.ShapeDtypeStruct((_N, _D), jnp.float32)
    )(degp, p, y, b, g, be, w)


def _tc_last(degp, p, y, b, g, be, wlin, blin):
    def body(degp_ref, p_ref, y_ref, b_ref, g_ref, be_ref, w_ref, bl_ref, out_ref):
        dinv = _dinv_from(degp_ref[...])
        pv = p_ref[...]
        t = (pv[0, 0:_N, :] + pv[1, 0:_N, :] + y_ref[...]) * dinv + b_ref[...]
        t = jnp.maximum(t, 0.0)
        m = jnp.mean(t, axis=0, keepdims=True)
        v = jnp.mean((t - m) * (t - m), axis=0, keepdims=True)
        h = g_ref[...] * (t - m) * lax.rsqrt(v + _EPS) + be_ref[...]
        out_ref[...] = (
            jnp.dot(h, w_ref[...], preferred_element_type=jnp.float32) + bl_ref[...]
        )

    return pl.pallas_call(
        body, out_shape=jax.ShapeDtypeStruct((_N, _D), jnp.float32)
    )(degp, p, y, b, g, be, wlin, blin)


def kernel(x, edge_index, W1, b1, W2, b2, W3, b3, g1, be1, g2, be2, g3, be3,
           Wlin, blin):
    src = edge_index[0].astype(jnp.int32)
    dst = edge_index[1].astype(jnp.int32)
    pad = _EPAD - _E
    srcp = jnp.concatenate([src, jnp.zeros((pad,), jnp.int32)])
    dstp = jnp.concatenate([dst, jnp.full((pad,), _N, jnp.int32)])

    degp = _deg_kernel(dstp)

    b1r, g1r, be1r = b1.reshape(1, _H), g1.reshape(1, _H), be1.reshape(1, _H)
    b2r, g2r, be2r = b2.reshape(1, _H), g2.reshape(1, _H), be2.reshape(1, _H)
    b3r, g3r, be3r = b3.reshape(1, _H), g3.reshape(1, _H), be3.reshape(1, _H)
    wlinp = jnp.zeros((_H, _D), jnp.float32).at[:, :_C].set(Wlin)
    blinp = jnp.zeros((1, _D), jnp.float32).at[0, :_C].set(blin)

    y1 = _tc_first(degp, x, W1)
    p1 = _scatter_kernel(y1, srcp, dstp)
    y2 = _tc_mid(degp, p1, y1, b1r, g1r, be1r, W2)
    p2 = _scatter_kernel(y2, srcp, dstp)
    y3 = _tc_mid(degp, p2, y2, b2r, g2r, be2r, W3)
    p3 = _scatter_kernel(y3, srcp, dstp)
    out = _tc_last(degp, p3, y3, b3r, g3r, be3r, wlinp, blinp)
    return out[:, :_C]
